# Initial kernel scaffold; baseline (speedup 1.0000x reference)
import jax
import jax.numpy as jnp
from jax.experimental import pallas as pl


def kernel(x, edge_attr, edge_index, v_w, v_b, e_w, e_b, bn_g, bn_b):
    raise NotImplementedError("write your pallas kernel here")


# speedup vs baseline: 2.3572x; 2.3572x over previous
import functools

import jax
import jax.numpy as jnp
from jax import lax
from jax.experimental import pallas as pl
from jax.experimental.pallas import tpu as pltpu
from jax.experimental.pallas import tpu_sc as plsc

N = 10000
E = 320000
D = 128
EPS = 1e-5

NC = 2
NS = 16
NW = NC * NS
EPT = E // NW
C = 40
NCH = EPT // C
IB = 25
NIT = NCH // IB
RPT = N // NS
RPB = 25
NDUMP = RPT // RPB
CW = 16

BN = 1000
BE = 4000

_HI = jax.lax.Precision.HIGHEST


def _vertex_body(x_ref, w_ref, b_ref, o1, o2, o3, o4):
    xb = x_ref[...]
    outs = (o1, o2, o3, o4)
    for k in range(4):
        outs[k][...] = jnp.dot(xb, w_ref[k], precision=_HI) + b_ref[k]


def _vertex_matmuls(x, vw, vb):
    nb = N // BN
    return pl.pallas_call(
        _vertex_body,
        grid=(nb,),
        in_specs=[
            pl.BlockSpec((BN, D), lambda i: (i, 0)),
            pl.BlockSpec((4, D, D), lambda i: (0, 0, 0)),
            pl.BlockSpec((4, 1, D), lambda i: (0, 0, 0)),
        ],
        out_specs=[pl.BlockSpec((BN, D), lambda i: (i, 0))] * 4,
        out_shape=[jax.ShapeDtypeStruct((N, D), jnp.float32)] * 4,
    )(x, vw, vb.reshape(4, 1, D))


def _sc_edge_body(w0_hbm, src_hbm, dst_hbm, x2_hbm, x3_hbm, x4_hbm,
                  g_hbm, agg_hbm, cnt_hbm,
                  idx_s, idx_d, wbuf, xg2, xg3, xg4, oneb,
                  zbuf, zbuf2, acc_sh, cnt_sh, sem):
    cid = lax.axis_index("c")
    sid = lax.axis_index("s")
    wid = sid * NC + cid

    zero16 = jnp.zeros((16,), jnp.float32)
    one16 = jnp.ones((16,), jnp.float32)

    def _zrow(r, _):
        for cc in range(D // 16):
            zbuf[r, pl.ds(cc * 16, 16)] = zero16
        zbuf2[r, pl.ds(0, 16)] = zero16
        return 0

    lax.fori_loop(0, RPB, _zrow, 0)

    def _orow(r, _):
        oneb[r, pl.ds(0, 16)] = one16
        return 0

    lax.fori_loop(0, C, _orow, 0)

    row0 = sid * RPT

    def _zero(j, _):
        pltpu.sync_copy(zbuf, acc_sh.at[pl.ds(row0 + j * RPB, RPB)])
        pltpu.sync_copy(zbuf2, cnt_sh.at[pl.ds(row0 + j * RPB, RPB)])
        return 0

    lax.fori_loop(0, NDUMP, _zero, 0)

    plsc.subcore_barrier()

    ebase = wid * EPT

    def _itile(t, _):
        pltpu.sync_copy(src_hbm.at[wid * NIT + t], idx_s)
        pltpu.sync_copy(dst_hbm.at[wid * NIT + t], idx_d)

        def _chunk(ci, _):
            eoff = ebase + (t * IB + ci) * C
            srow = idx_s.at[ci]
            drow = idx_d.at[ci]
            c2 = pltpu.async_copy(x2_hbm.at[drow], xg2, sem)
            c3 = pltpu.async_copy(x3_hbm.at[srow], xg3, sem)
            c4 = pltpu.async_copy(x4_hbm.at[drow], xg4, sem)
            pltpu.sync_copy(w0_hbm.at[pl.ds(eoff, C)], wbuf)
            c2.wait()
            c3.wait()
            c4.wait()

            def _row(r, _):
                for cc in range(D // 16):
                    sl = pl.ds(cc * 16, 16)
                    wv = wbuf[r, sl]
                    sg = 1.0 / (1.0 + jnp.exp(-wv))
                    wbuf[r, sl] = sg * xg2[r, sl]
                    xg3[r, sl] = xg3[r, sl] + xg4[r, sl]
                return 0

            lax.fori_loop(0, C, _row, 0)

            pltpu.sync_copy(xg3, g_hbm.at[pl.ds(eoff, C)])
            pltpu.sync_copy(wbuf, acc_sh.at[srow], add=True)
            pltpu.sync_copy(oneb, cnt_sh.at[srow], add=True)
            return 0

        lax.fori_loop(0, IB, _chunk, 0)
        return 0

    lax.fori_loop(0, NIT, _itile, 0)

    plsc.subcore_barrier()

    obase = cid * N + row0

    def _dump(j, _):
        pltpu.sync_copy(acc_sh.at[pl.ds(row0 + j * RPB, RPB)], zbuf)
        pltpu.sync_copy(zbuf, agg_hbm.at[pl.ds(obase + j * RPB, RPB)])
        pltpu.sync_copy(cnt_sh.at[pl.ds(row0 + j * RPB, RPB)], zbuf2)
        pltpu.sync_copy(zbuf2, cnt_hbm.at[pl.ds(obase + j * RPB, RPB)])
        return 0

    lax.fori_loop(0, NDUMP, _dump, 0)


def _sc_edge(w0, src4, dst4, x2, x3, x4):
    mesh = plsc.VectorSubcoreMesh(core_axis_name="c", subcore_axis_name="s",
                                  num_cores=NC, num_subcores=NS)
    f = pl.kernel(
        _sc_edge_body,
        out_type=(jax.ShapeDtypeStruct((E, D), jnp.float32),
                  jax.ShapeDtypeStruct((2 * N, D), jnp.float32),
                  jax.ShapeDtypeStruct((2 * N, CW), jnp.float32)),
        mesh=mesh,
        scratch_types=[
            pltpu.VMEM((IB, C), jnp.int32),
            pltpu.VMEM((IB, C), jnp.int32),
            pltpu.VMEM((C, D), jnp.float32),
            pltpu.VMEM((C, D), jnp.float32),
            pltpu.VMEM((C, D), jnp.float32),
            pltpu.VMEM((C, D), jnp.float32),
            pltpu.VMEM((C, CW), jnp.float32),
            pltpu.VMEM((RPB, D), jnp.float32),
            pltpu.VMEM((RPB, CW), jnp.float32),
            pltpu.VMEM_SHARED((N, D), jnp.float32),
            pltpu.VMEM_SHARED((N, CW), jnp.float32),
            pltpu.SemaphoreType.DMA,
        ],
        compiler_params=pltpu.CompilerParams(use_tc_tiling_on_sc=False),
    )
    return f(w0, src4, dst4, x2, x3, x4)


def _node_body(x0r, x1r, p0r, p1r, c0r, c1r, gr, br, outr, ssum, ssq):
    ph = pl.program_id(0)

    @pl.when((ph == 0) & (pl.program_id(1) == 0))
    def _():
        ssum[...] = jnp.zeros_like(ssum)
        ssq[...] = jnp.zeros_like(ssq)

    cnt = c0r[...][:, 0:1] + c1r[...][:, 0:1]
    z = x1r[...] + (p0r[...] + p1r[...]) / jnp.maximum(cnt, 1.0)

    @pl.when(ph == 0)
    def _():
        ssum[...] += jnp.sum(z, axis=0, keepdims=True)
        ssq[...] += jnp.sum(z * z, axis=0, keepdims=True)

    @pl.when(ph == 1)
    def _():
        mu = ssum[...] * (1.0 / N)
        var = ssq[...] * (1.0 / N) - mu * mu
        y = (z - mu) * lax.rsqrt(var + EPS) * gr[...] + br[...]
        outr[...] = x0r[...] + y / (1.0 + jnp.exp(-y))


def _node_update(x0, x1, p, pc, g, b):
    nb = N // BN
    return pl.pallas_call(
        _node_body,
        grid=(2, nb),
        in_specs=[
            pl.BlockSpec((BN, D), lambda ph, i: (i, 0)),
            pl.BlockSpec((BN, D), lambda ph, i: (i, 0)),
            pl.BlockSpec((BN, D), lambda ph, i: (i, 0)),
            pl.BlockSpec((BN, D), lambda ph, i: (i + nb, 0)),
            pl.BlockSpec((BN, CW), lambda ph, i: (i, 0)),
            pl.BlockSpec((BN, CW), lambda ph, i: (i + nb, 0)),
            pl.BlockSpec((1, D), lambda ph, i: (0, 0)),
            pl.BlockSpec((1, D), lambda ph, i: (0, 0)),
        ],
        out_specs=pl.BlockSpec((BN, D), lambda ph, i: (i * ph, 0)),
        out_shape=jax.ShapeDtypeStruct((N, D), jnp.float32),
        scratch_shapes=[pltpu.VMEM((1, D), jnp.float32),
                        pltpu.VMEM((1, D), jnp.float32)],
        compiler_params=pltpu.CompilerParams(
            dimension_semantics=("arbitrary", "arbitrary")),
    )(x0, x1, p, p, pc, pc, g.reshape(1, D), b.reshape(1, D))


def _edge_body(w0r, gr, ewr, ebr, bgr, bbr, outr, ssum, ssq):
    ph = pl.program_id(0)

    @pl.when((ph == 0) & (pl.program_id(1) == 0))
    def _():
        ssum[...] = jnp.zeros_like(ssum)
        ssq[...] = jnp.zeros_like(ssq)

    z = jnp.dot(w0r[...], ewr[...], precision=_HI) + ebr[...] + gr[...]

    @pl.when(ph == 0)
    def _():
        ssum[...] += jnp.sum(z, axis=0, keepdims=True)
        ssq[...] += jnp.sum(z * z, axis=0, keepdims=True)

    @pl.when(ph == 1)
    def _():
        mu = ssum[...] * (1.0 / E)
        var = ssq[...] * (1.0 / E) - mu * mu
        y = (z - mu) * lax.rsqrt(var + EPS) * bgr[...] + bbr[...]
        outr[...] = w0r[...] + y / (1.0 + jnp.exp(-y))


def _edge_update(w0, g, ew, eb, bg, bb):
    nbe = E // BE
    return pl.pallas_call(
        _edge_body,
        grid=(2, nbe),
        in_specs=[
            pl.BlockSpec((BE, D), lambda ph, i: (i, 0)),
            pl.BlockSpec((BE, D), lambda ph, i: (i, 0)),
            pl.BlockSpec((D, D), lambda ph, i: (0, 0)),
            pl.BlockSpec((1, D), lambda ph, i: (0, 0)),
            pl.BlockSpec((1, D), lambda ph, i: (0, 0)),
            pl.BlockSpec((1, D), lambda ph, i: (0, 0)),
        ],
        out_specs=pl.BlockSpec((BE, D), lambda ph, i: (i * ph, 0)),
        out_shape=jax.ShapeDtypeStruct((E, D), jnp.float32),
        scratch_shapes=[pltpu.VMEM((1, D), jnp.float32),
                        pltpu.VMEM((1, D), jnp.float32)],
        compiler_params=pltpu.CompilerParams(
            dimension_semantics=("arbitrary", "arbitrary")),
    )(w0, g, ew, eb.reshape(1, D), bg.reshape(1, D), bb.reshape(1, D))


def kernel(x, edge_attr, edge_index, v_w, v_b, e_w, e_b, bn_g, bn_b):
    src4 = edge_index[0].reshape(NW * NIT, IB, C)
    dst4 = edge_index[1].reshape(NW * NIT, IB, C)
    w = edge_attr
    for l in range(v_w.shape[0]):
        x1, x2, x3, x4 = _vertex_matmuls(x, v_w[l], v_b[l])
        g, p, pc = _sc_edge(w, src4, dst4, x2, x3, x4)
        x = _node_update(x, x1, p, pc, bn_g[l, 0], bn_b[l, 0])
        w = _edge_update(w, g, e_w[l], e_b[l], bn_g[l, 1], bn_b[l, 1])
    return x, w


# speedup vs baseline: 3.2501x; 1.3788x over previous
import functools

import jax
import jax.numpy as jnp
from jax import lax
from jax.experimental import pallas as pl
from jax.experimental.pallas import tpu as pltpu
from jax.experimental.pallas import tpu_sc as plsc

N = 10000
E = 320000
D = 128
EPS = 1e-5

NC = 2
NS = 16
NW = NC * NS
EPT = E // NW
C = 40
NCH = EPT // C
IB = 50
NIT = NCH // IB
RPT = N // NS
RPB = 25
NDUMP = RPT // RPB
CW = 16
HC = 125
NHC = EPT // HC

BN = 1000
BE = 4000

_HI = jax.lax.Precision.HIGHEST


def _vertex_body(x_ref, w_ref, b_ref, o1, o2, o3, o4):
    xb = x_ref[...]
    outs = (o1, o2, o3, o4)
    for k in range(4):
        outs[k][...] = jnp.dot(xb, w_ref[k], precision=_HI) + b_ref[k]


def _vertex_matmuls(x, vw, vb):
    nb = N // BN
    return pl.pallas_call(
        _vertex_body,
        grid=(nb,),
        in_specs=[
            pl.BlockSpec((BN, D), lambda i: (i, 0)),
            pl.BlockSpec((4, D, D), lambda i: (0, 0, 0)),
            pl.BlockSpec((4, 1, D), lambda i: (0, 0, 0)),
        ],
        out_specs=[pl.BlockSpec((BN, D), lambda i: (i, 0))] * 4,
        out_shape=[jax.ShapeDtypeStruct((N, D), jnp.float32)] * 4,
    )(x, vw, vb.reshape(4, 1, D))


def _sc_edge_body(w0_hbm, src_hbm, dst_hbm, x2_hbm, x3_hbm, x4_hbm,
                  g_hbm, agg_hbm,
                  idx_s, idx_d,
                  wbuf0, xg20, xg30, xg40,
                  wbuf1, xg21, xg31, xg41,
                  acc_sh, sem0, sem1):
    cid = lax.axis_index("c")
    sid = lax.axis_index("s")
    wid = sid * NC + cid
    row0 = sid * RPT
    ebase = wid * EPT

    wbufs = (wbuf0, wbuf1)
    xg2s = (xg20, xg21)
    xg3s = (xg30, xg31)
    xg4s = (xg40, xg41)
    sems = (sem0, sem1)

    zero16 = jnp.zeros((16,), jnp.float32)

    def _zrow(r, _):
        for cc in range(D // 16):
            xg20[r, pl.ds(cc * 16, 16)] = zero16
        return 0

    lax.fori_loop(0, RPB, _zrow, 0)

    def _zero(j, _):
        pltpu.sync_copy(xg20.at[pl.ds(0, RPB)],
                        acc_sh.at[pl.ds(row0 + j * RPB, RPB)])
        return 0

    lax.fori_loop(0, NDUMP, _zero, 0)

    plsc.subcore_barrier()

    def _issue(b, t, ci):
        srow = idx_s.at[ci]
        drow = idx_d.at[ci]
        pltpu.async_copy(x2_hbm.at[drow], xg2s[b], sems[b])
        pltpu.async_copy(x3_hbm.at[srow], xg3s[b], sems[b])
        pltpu.async_copy(x4_hbm.at[drow], xg4s[b], sems[b])
        eoff = ebase + (t * IB + ci) * C
        pltpu.async_copy(w0_hbm.at[pl.ds(eoff, C)], wbufs[b], sems[b])

    def _process(b, t, ci):
        wbuf, xg2, xg3, xg4 = wbufs[b], xg2s[b], xg3s[b], xg4s[b]
        for dst in (xg2, xg3, xg4, wbuf):
            pltpu.make_async_copy(w0_hbm.at[pl.ds(0, C)], dst, sems[b]).wait()

        def _row(r, _):
            for cc in range(D // 16):
                sl = pl.ds(cc * 16, 16)
                wv = wbuf[r, sl]
                sg = 1.0 / (1.0 + jnp.exp(-wv))
                wbuf[r, sl] = sg * xg2[r, sl]
                xg3[r, sl] = xg3[r, sl] + xg4[r, sl]
            return 0

        lax.fori_loop(0, C, _row, 0)

        eoff = ebase + (t * IB + ci) * C
        pltpu.sync_copy(xg3, g_hbm.at[pl.ds(eoff, C)])
        pltpu.sync_copy(wbuf, acc_sh.at[idx_s.at[ci]], add=True)

    def _itile(t, _):
        pltpu.sync_copy(src_hbm.at[wid * NIT + t], idx_s)
        pltpu.sync_copy(dst_hbm.at[wid * NIT + t], idx_d)
        _issue(0, t, 0)
        _issue(1, t, 1)

        def _pair(j, _):
            for b in range(2):
                ci = 2 * j + b
                _process(b, t, ci)
                _issue(b, t, ci + 2)
            return 0

        lax.fori_loop(0, IB // 2 - 1, _pair, 0)
        _process(0, t, IB - 2)
        _process(1, t, IB - 1)
        return 0

    lax.fori_loop(0, NIT, _itile, 0)

    plsc.subcore_barrier()

    obase = cid * N + row0

    def _dump(j, _):
        pltpu.sync_copy(acc_sh.at[pl.ds(row0 + j * RPB, RPB)],
                        xg20.at[pl.ds(0, RPB)])
        pltpu.sync_copy(xg20.at[pl.ds(0, RPB)],
                        agg_hbm.at[pl.ds(obase + j * RPB, RPB)])
        return 0

    lax.fori_loop(0, NDUMP, _dump, 0)


def _sc_edge(w0, src4, dst4, x2, x3, x4):
    mesh = plsc.VectorSubcoreMesh(core_axis_name="c", subcore_axis_name="s",
                                  num_cores=NC, num_subcores=NS)
    f = pl.kernel(
        _sc_edge_body,
        out_type=(jax.ShapeDtypeStruct((E, D), jnp.float32),
                  jax.ShapeDtypeStruct((2 * N, D), jnp.float32)),
        mesh=mesh,
        scratch_types=[
            pltpu.VMEM((IB, C), jnp.int32),
            pltpu.VMEM((IB, C), jnp.int32),
            pltpu.VMEM((C, D), jnp.float32),
            pltpu.VMEM((C, D), jnp.float32),
            pltpu.VMEM((C, D), jnp.float32),
            pltpu.VMEM((C, D), jnp.float32),
            pltpu.VMEM((C, D), jnp.float32),
            pltpu.VMEM((C, D), jnp.float32),
            pltpu.VMEM((C, D), jnp.float32),
            pltpu.VMEM((C, D), jnp.float32),
            pltpu.VMEM_SHARED((N, D), jnp.float32),
            pltpu.SemaphoreType.DMA,
            pltpu.SemaphoreType.DMA,
        ],
        compiler_params=pltpu.CompilerParams(use_tc_tiling_on_sc=False),
    )
    return f(w0, src4, dst4, x2, x3, x4)


def _sc_cnt_body(src_hbm, cnt_hbm, idxb, oneb, zb, cnt_sh):
    cid = lax.axis_index("c")
    sid = lax.axis_index("s")
    wid = sid * NC + cid
    row0 = sid * RPT

    pltpu.sync_copy(src_hbm.at[wid], idxb)

    zero16 = jnp.zeros((16,), jnp.float32)
    one16 = jnp.ones((16,), jnp.float32)

    def _orow(r, _):
        oneb[r, pl.ds(0, 16)] = one16
        return 0

    lax.fori_loop(0, HC, _orow, 0)

    def _zrow(r, _):
        zb[r, pl.ds(0, 16)] = zero16
        return 0

    lax.fori_loop(0, RPB, _zrow, 0)

    def _zero(j, _):
        pltpu.sync_copy(zb, cnt_sh.at[pl.ds(row0 + j * RPB, RPB)])
        return 0

    lax.fori_loop(0, NDUMP, _zero, 0)

    plsc.subcore_barrier()

    def _chunk(k, _):
        pltpu.sync_copy(oneb, cnt_sh.at[idxb.at[k]], add=True)
        return 0

    lax.fori_loop(0, NHC, _chunk, 0)

    plsc.subcore_barrier()

    obase = cid * N + row0

    def _dump(j, _):
        pltpu.sync_copy(cnt_sh.at[pl.ds(row0 + j * RPB, RPB)], zb)
        pltpu.sync_copy(zb, cnt_hbm.at[pl.ds(obase + j * RPB, RPB)])
        return 0

    lax.fori_loop(0, NDUMP, _dump, 0)


def _sc_cnt(src5):
    mesh = plsc.VectorSubcoreMesh(core_axis_name="c", subcore_axis_name="s",
                                  num_cores=NC, num_subcores=NS)
    f = pl.kernel(
        _sc_cnt_body,
        out_type=jax.ShapeDtypeStruct((2 * N, CW), jnp.float32),
        mesh=mesh,
        scratch_types=[
            pltpu.VMEM((NHC, HC), jnp.int32),
            pltpu.VMEM((HC, CW), jnp.float32),
            pltpu.VMEM((RPB, CW), jnp.float32),
            pltpu.VMEM_SHARED((N, CW), jnp.float32),
        ],
        compiler_params=pltpu.CompilerParams(use_tc_tiling_on_sc=False),
    )
    return f(src5)


def _node_body(x0r, x1r, p0r, p1r, c0r, c1r, gr, br, outr, ssum, ssq):
    ph = pl.program_id(0)

    @pl.when((ph == 0) & (pl.program_id(1) == 0))
    def _():
        ssum[...] = jnp.zeros_like(ssum)
        ssq[...] = jnp.zeros_like(ssq)

    cnt = c0r[...][:, 0:1] + c1r[...][:, 0:1]
    z = x1r[...] + (p0r[...] + p1r[...]) / jnp.maximum(cnt, 1.0)

    @pl.when(ph == 0)
    def _():
        ssum[...] += jnp.sum(z, axis=0, keepdims=True)
        ssq[...] += jnp.sum(z * z, axis=0, keepdims=True)

    @pl.when(ph == 1)
    def _():
        mu = ssum[...] * (1.0 / N)
        var = ssq[...] * (1.0 / N) - mu * mu
        y = (z - mu) * lax.rsqrt(var + EPS) * gr[...] + br[...]
        outr[...] = x0r[...] + y / (1.0 + jnp.exp(-y))


def _node_update(x0, x1, p, pc, g, b):
    nb = N // BN
    return pl.pallas_call(
        _node_body,
        grid=(2, nb),
        in_specs=[
            pl.BlockSpec((BN, D), lambda ph, i: (i, 0)),
            pl.BlockSpec((BN, D), lambda ph, i: (i, 0)),
            pl.BlockSpec((BN, D), lambda ph, i: (i, 0)),
            pl.BlockSpec((BN, D), lambda ph, i: (i + nb, 0)),
            pl.BlockSpec((BN, CW), lambda ph, i: (i, 0)),
            pl.BlockSpec((BN, CW), lambda ph, i: (i + nb, 0)),
            pl.BlockSpec((1, D), lambda ph, i: (0, 0)),
            pl.BlockSpec((1, D), lambda ph, i: (0, 0)),
        ],
        out_specs=pl.BlockSpec((BN, D), lambda ph, i: (i * ph, 0)),
        out_shape=jax.ShapeDtypeStruct((N, D), jnp.float32),
        scratch_shapes=[pltpu.VMEM((1, D), jnp.float32),
                        pltpu.VMEM((1, D), jnp.float32)],
        compiler_params=pltpu.CompilerParams(
            dimension_semantics=("arbitrary", "arbitrary")),
    )(x0, x1, p, p, pc, pc, g.reshape(1, D), b.reshape(1, D))


def _edge_body(w0r, gr, ewr, ebr, bgr, bbr, outr, ssum, ssq):
    ph = pl.program_id(0)

    @pl.when((ph == 0) & (pl.program_id(1) == 0))
    def _():
        ssum[...] = jnp.zeros_like(ssum)
        ssq[...] = jnp.zeros_like(ssq)

    z = jnp.dot(w0r[...], ewr[...], precision=_HI) + ebr[...] + gr[...]

    @pl.when(ph == 0)
    def _():
        ssum[...] += jnp.sum(z, axis=0, keepdims=True)
        ssq[...] += jnp.sum(z * z, axis=0, keepdims=True)

    @pl.when(ph == 1)
    def _():
        mu = ssum[...] * (1.0 / E)
        var = ssq[...] * (1.0 / E) - mu * mu
        y = (z - mu) * lax.rsqrt(var + EPS) * bgr[...] + bbr[...]
        outr[...] = w0r[...] + y / (1.0 + jnp.exp(-y))


def _edge_update(w0, g, ew, eb, bg, bb):
    nbe = E // BE
    return pl.pallas_call(
        _edge_body,
        grid=(2, nbe),
        in_specs=[
            pl.BlockSpec((BE, D), lambda ph, i: (i, 0)),
            pl.BlockSpec((BE, D), lambda ph, i: (i, 0)),
            pl.BlockSpec((D, D), lambda ph, i: (0, 0)),
            pl.BlockSpec((1, D), lambda ph, i: (0, 0)),
            pl.BlockSpec((1, D), lambda ph, i: (0, 0)),
            pl.BlockSpec((1, D), lambda ph, i: (0, 0)),
        ],
        out_specs=pl.BlockSpec((BE, D), lambda ph, i: (i * ph, 0)),
        out_shape=jax.ShapeDtypeStruct((E, D), jnp.float32),
        scratch_shapes=[pltpu.VMEM((1, D), jnp.float32),
                        pltpu.VMEM((1, D), jnp.float32)],
        compiler_params=pltpu.CompilerParams(
            dimension_semantics=("arbitrary", "arbitrary")),
    )(w0, g, ew, eb.reshape(1, D), bg.reshape(1, D), bb.reshape(1, D))


def kernel(x, edge_attr, edge_index, v_w, v_b, e_w, e_b, bn_g, bn_b):
    src4 = edge_index[0].reshape(NW * NIT, IB, C)
    dst4 = edge_index[1].reshape(NW * NIT, IB, C)
    src5 = edge_index[0].reshape(NW, NHC, HC)
    pc = _sc_cnt(src5)
    w = edge_attr
    for l in range(v_w.shape[0]):
        x1, x2, x3, x4 = _vertex_matmuls(x, v_w[l], v_b[l])
        g, p = _sc_edge(w, src4, dst4, x2, x3, x4)
        x = _node_update(x, x1, p, pc, bn_g[l, 0], bn_b[l, 0])
        w = _edge_update(w, g, e_w[l], e_b[l], bn_g[l, 1], bn_b[l, 1])
    return x, w


# speedup vs baseline: 3.8459x; 1.1833x over previous
import functools

import jax
import jax.numpy as jnp
from jax import lax
from jax.experimental import pallas as pl
from jax.experimental.pallas import tpu as pltpu
from jax.experimental.pallas import tpu_sc as plsc

N = 10000
E = 320000
D = 128
EPS = 1e-5

NC = 2
NS = 16
NW = NC * NS
EPT = E // NW
C = 40
NCH = EPT // C
IB = 50
NIT = NCH // IB
RPT = N // NS
RPB = 25
NDUMP = RPT // RPB
CW = 16
HC = 125
NHC = EPT // HC

BN = 1000
BE = 8000

def _dot3(a, b):
    ah = a.astype(jnp.bfloat16)
    al = (a - ah.astype(jnp.float32)).astype(jnp.bfloat16)
    bh = b.astype(jnp.bfloat16)
    bl = (b - bh.astype(jnp.float32)).astype(jnp.bfloat16)
    d = functools.partial(
        jax.lax.dot_general,
        dimension_numbers=(((1,), (0,)), ((), ())),
        preferred_element_type=jnp.float32)
    return d(ah, bh) + (d(ah, bl) + d(al, bh))


def _vertex_body(x_ref, w_ref, b_ref, o1, o2, o3, o4):
    xb = x_ref[...]
    outs = (o1, o2, o3, o4)
    for k in range(4):
        outs[k][...] = _dot3(xb, w_ref[k]) + b_ref[k]


def _vertex_matmuls(x, vw, vb):
    nb = N // BN
    return pl.pallas_call(
        _vertex_body,
        grid=(nb,),
        in_specs=[
            pl.BlockSpec((BN, D), lambda i: (i, 0)),
            pl.BlockSpec((4, D, D), lambda i: (0, 0, 0)),
            pl.BlockSpec((4, 1, D), lambda i: (0, 0, 0)),
        ],
        out_specs=[pl.BlockSpec((BN, D), lambda i: (i, 0))] * 4,
        out_shape=[jax.ShapeDtypeStruct((N, D), jnp.float32)] * 4,
    )(x, vw, vb.reshape(4, 1, D))


def _sc_edge_body(w0_hbm, src_hbm, dst_hbm, x2_hbm, x3_hbm, x4_hbm,
                  g_hbm, agg_hbm,
                  idx_s, idx_d,
                  wbuf0, xg20, xg30, xg40,
                  wbuf1, xg21, xg31, xg41,
                  acc_sh, sem0, sem1):
    cid = lax.axis_index("c")
    sid = lax.axis_index("s")
    wid = sid * NC + cid
    row0 = sid * RPT
    ebase = wid * EPT

    wbufs = (wbuf0, wbuf1)
    xg2s = (xg20, xg21)
    xg3s = (xg30, xg31)
    xg4s = (xg40, xg41)
    sems = (sem0, sem1)

    zero16 = jnp.zeros((16,), jnp.float32)

    def _zrow(r, _):
        for cc in range(D // 16):
            xg20[r, pl.ds(cc * 16, 16)] = zero16
        return 0

    lax.fori_loop(0, RPB, _zrow, 0)

    def _zero(j, _):
        pltpu.sync_copy(xg20.at[pl.ds(0, RPB)],
                        acc_sh.at[pl.ds(row0 + j * RPB, RPB)])
        return 0

    lax.fori_loop(0, NDUMP, _zero, 0)

    plsc.subcore_barrier()

    def _issue(b, t, ci):
        srow = idx_s.at[ci]
        drow = idx_d.at[ci]
        pltpu.async_copy(x2_hbm.at[drow], xg2s[b], sems[b])
        pltpu.async_copy(x3_hbm.at[srow], xg3s[b], sems[b])
        pltpu.async_copy(x4_hbm.at[drow], xg4s[b], sems[b])
        eoff = ebase + (t * IB + ci) * C
        pltpu.async_copy(w0_hbm.at[pl.ds(eoff, C)], wbufs[b], sems[b])

    def _process(b, t, ci):
        wbuf, xg2, xg3, xg4 = wbufs[b], xg2s[b], xg3s[b], xg4s[b]
        for dst in (xg2, xg3, xg4, wbuf):
            pltpu.make_async_copy(w0_hbm.at[pl.ds(0, C)], dst, sems[b]).wait()

        def _row(r, _):
            for cc in range(D // 16):
                sl = pl.ds(cc * 16, 16)
                wv = wbuf[r, sl]
                sg = 1.0 / (1.0 + jnp.exp(-wv))
                wbuf[r, sl] = sg * xg2[r, sl]
                xg3[r, sl] = xg3[r, sl] + xg4[r, sl]
            return 0

        lax.fori_loop(0, C, _row, 0)

        eoff = ebase + (t * IB + ci) * C
        pltpu.sync_copy(xg3, g_hbm.at[pl.ds(eoff, C)])
        pltpu.sync_copy(wbuf, acc_sh.at[idx_s.at[ci]], add=True)

    def _itile(t, _):
        pltpu.sync_copy(src_hbm.at[wid * NIT + t], idx_s)
        pltpu.sync_copy(dst_hbm.at[wid * NIT + t], idx_d)
        _issue(0, t, 0)
        _issue(1, t, 1)

        def _pair(j, _):
            for b in range(2):
                ci = 2 * j + b
                _process(b, t, ci)
                _issue(b, t, ci + 2)
            return 0

        lax.fori_loop(0, IB // 2 - 1, _pair, 0)
        _process(0, t, IB - 2)
        _process(1, t, IB - 1)
        return 0

    lax.fori_loop(0, NIT, _itile, 0)

    plsc.subcore_barrier()

    obase = cid * N + row0

    def _dump(j, _):
        pltpu.sync_copy(acc_sh.at[pl.ds(row0 + j * RPB, RPB)],
                        xg20.at[pl.ds(0, RPB)])
        pltpu.sync_copy(xg20.at[pl.ds(0, RPB)],
                        agg_hbm.at[pl.ds(obase + j * RPB, RPB)])
        return 0

    lax.fori_loop(0, NDUMP, _dump, 0)


def _sc_edge(w0, src4, dst4, x2, x3, x4):
    mesh = plsc.VectorSubcoreMesh(core_axis_name="c", subcore_axis_name="s",
                                  num_cores=NC, num_subcores=NS)
    f = pl.kernel(
        _sc_edge_body,
        out_type=(jax.ShapeDtypeStruct((E, D), jnp.float32),
                  jax.ShapeDtypeStruct((2 * N, D), jnp.float32)),
        mesh=mesh,
        scratch_types=[
            pltpu.VMEM((IB, C), jnp.int32),
            pltpu.VMEM((IB, C), jnp.int32),
            pltpu.VMEM((C, D), jnp.float32),
            pltpu.VMEM((C, D), jnp.float32),
            pltpu.VMEM((C, D), jnp.float32),
            pltpu.VMEM((C, D), jnp.float32),
            pltpu.VMEM((C, D), jnp.float32),
            pltpu.VMEM((C, D), jnp.float32),
            pltpu.VMEM((C, D), jnp.float32),
            pltpu.VMEM((C, D), jnp.float32),
            pltpu.VMEM_SHARED((N, D), jnp.float32),
            pltpu.SemaphoreType.DMA,
            pltpu.SemaphoreType.DMA,
        ],
        compiler_params=pltpu.CompilerParams(use_tc_tiling_on_sc=False),
    )
    return f(w0, src4, dst4, x2, x3, x4)


def _sc_cnt_body(src_hbm, cnt_hbm, idxb, oneb, zb, cnt_sh):
    cid = lax.axis_index("c")
    sid = lax.axis_index("s")
    wid = sid * NC + cid
    row0 = sid * RPT

    pltpu.sync_copy(src_hbm.at[wid], idxb)

    zero16 = jnp.zeros((16,), jnp.float32)
    one16 = jnp.ones((16,), jnp.float32)

    def _orow(r, _):
        oneb[r, pl.ds(0, 16)] = one16
        return 0

    lax.fori_loop(0, HC, _orow, 0)

    def _zrow(r, _):
        zb[r, pl.ds(0, 16)] = zero16
        return 0

    lax.fori_loop(0, RPB, _zrow, 0)

    def _zero(j, _):
        pltpu.sync_copy(zb, cnt_sh.at[pl.ds(row0 + j * RPB, RPB)])
        return 0

    lax.fori_loop(0, NDUMP, _zero, 0)

    plsc.subcore_barrier()

    def _chunk(k, _):
        pltpu.sync_copy(oneb, cnt_sh.at[idxb.at[k]], add=True)
        return 0

    lax.fori_loop(0, NHC, _chunk, 0)

    plsc.subcore_barrier()

    obase = cid * N + row0

    def _dump(j, _):
        pltpu.sync_copy(cnt_sh.at[pl.ds(row0 + j * RPB, RPB)], zb)
        pltpu.sync_copy(zb, cnt_hbm.at[pl.ds(obase + j * RPB, RPB)])
        return 0

    lax.fori_loop(0, NDUMP, _dump, 0)


def _sc_cnt(src5):
    mesh = plsc.VectorSubcoreMesh(core_axis_name="c", subcore_axis_name="s",
                                  num_cores=NC, num_subcores=NS)
    f = pl.kernel(
        _sc_cnt_body,
        out_type=jax.ShapeDtypeStruct((2 * N, CW), jnp.float32),
        mesh=mesh,
        scratch_types=[
            pltpu.VMEM((NHC, HC), jnp.int32),
            pltpu.VMEM((HC, CW), jnp.float32),
            pltpu.VMEM((RPB, CW), jnp.float32),
            pltpu.VMEM_SHARED((N, CW), jnp.float32),
        ],
        compiler_params=pltpu.CompilerParams(use_tc_tiling_on_sc=False),
    )
    return f(src5)


def _node_body(x0r, x1r, p0r, p1r, c0r, c1r, gr, br, outr, ssum, ssq):
    ph = pl.program_id(0)

    @pl.when((ph == 0) & (pl.program_id(1) == 0))
    def _():
        ssum[...] = jnp.zeros_like(ssum)
        ssq[...] = jnp.zeros_like(ssq)

    cnt = c0r[...][:, 0:1] + c1r[...][:, 0:1]
    z = x1r[...] + (p0r[...] + p1r[...]) / jnp.maximum(cnt, 1.0)

    @pl.when(ph == 0)
    def _():
        ssum[...] += jnp.sum(z, axis=0, keepdims=True)
        ssq[...] += jnp.sum(z * z, axis=0, keepdims=True)

    @pl.when(ph == 1)
    def _():
        mu = ssum[...] * (1.0 / N)
        var = ssq[...] * (1.0 / N) - mu * mu
        y = (z - mu) * lax.rsqrt(var + EPS) * gr[...] + br[...]
        outr[...] = x0r[...] + y / (1.0 + jnp.exp(-y))


def _node_update(x0, x1, p, pc, g, b):
    nb = N // BN
    return pl.pallas_call(
        _node_body,
        grid=(2, nb),
        in_specs=[
            pl.BlockSpec((BN, D), lambda ph, i: (i, 0)),
            pl.BlockSpec((BN, D), lambda ph, i: (i, 0)),
            pl.BlockSpec((BN, D), lambda ph, i: (i, 0)),
            pl.BlockSpec((BN, D), lambda ph, i: (i + nb, 0)),
            pl.BlockSpec((BN, CW), lambda ph, i: (i, 0)),
            pl.BlockSpec((BN, CW), lambda ph, i: (i + nb, 0)),
            pl.BlockSpec((1, D), lambda ph, i: (0, 0)),
            pl.BlockSpec((1, D), lambda ph, i: (0, 0)),
        ],
        out_specs=pl.BlockSpec((BN, D), lambda ph, i: (i * ph, 0)),
        out_shape=jax.ShapeDtypeStruct((N, D), jnp.float32),
        scratch_shapes=[pltpu.VMEM((1, D), jnp.float32),
                        pltpu.VMEM((1, D), jnp.float32)],
        compiler_params=pltpu.CompilerParams(
            dimension_semantics=("arbitrary", "arbitrary")),
    )(x0, x1, p, p, pc, pc, g.reshape(1, D), b.reshape(1, D))


def _edge_body(w0r, gr, ewr, ebr, bgr, bbr, outr, ssum, ssq):
    ph = pl.program_id(0)

    @pl.when((ph == 0) & (pl.program_id(1) == 0))
    def _():
        ssum[...] = jnp.zeros_like(ssum)
        ssq[...] = jnp.zeros_like(ssq)

    z = _dot3(w0r[...], ewr[...]) + ebr[...] + gr[...]

    @pl.when(ph == 0)
    def _():
        ssum[...] += jnp.sum(z, axis=0, keepdims=True)
        ssq[...] += jnp.sum(z * z, axis=0, keepdims=True)

    @pl.when(ph == 1)
    def _():
        mu = ssum[...] * (1.0 / E)
        var = ssq[...] * (1.0 / E) - mu * mu
        y = (z - mu) * lax.rsqrt(var + EPS) * bgr[...] + bbr[...]
        outr[...] = w0r[...] + y / (1.0 + jnp.exp(-y))


def _edge_update(w0, g, ew, eb, bg, bb):
    nbe = E // BE
    return pl.pallas_call(
        _edge_body,
        grid=(2, nbe),
        in_specs=[
            pl.BlockSpec((BE, D), lambda ph, i: (i, 0)),
            pl.BlockSpec((BE, D), lambda ph, i: (i, 0)),
            pl.BlockSpec((D, D), lambda ph, i: (0, 0)),
            pl.BlockSpec((1, D), lambda ph, i: (0, 0)),
            pl.BlockSpec((1, D), lambda ph, i: (0, 0)),
            pl.BlockSpec((1, D), lambda ph, i: (0, 0)),
        ],
        out_specs=pl.BlockSpec((BE, D), lambda ph, i: (i * ph, 0)),
        out_shape=jax.ShapeDtypeStruct((E, D), jnp.float32),
        scratch_shapes=[pltpu.VMEM((1, D), jnp.float32),
                        pltpu.VMEM((1, D), jnp.float32)],
        compiler_params=pltpu.CompilerParams(
            dimension_semantics=("arbitrary", "arbitrary")),
    )(w0, g, ew, eb.reshape(1, D), bg.reshape(1, D), bb.reshape(1, D))


def kernel(x, edge_attr, edge_index, v_w, v_b, e_w, e_b, bn_g, bn_b):
    src4 = edge_index[0].reshape(NW * NIT, IB, C)
    dst4 = edge_index[1].reshape(NW * NIT, IB, C)
    src5 = edge_index[0].reshape(NW, NHC, HC)
    pc = _sc_cnt(src5)
    w = edge_attr
    for l in range(v_w.shape[0]):
        x1, x2, x3, x4 = _vertex_matmuls(x, v_w[l], v_b[l])
        g, p = _sc_edge(w, src4, dst4, x2, x3, x4)
        x = _node_update(x, x1, p, pc, bn_g[l, 0], bn_b[l, 0])
        w = _edge_update(w, g, e_w[l], e_b[l], bn_g[l, 1], bn_b[l, 1])
    return x, w
